# baseline (speedup 1.0000x reference)
import functools

import jax
import jax.numpy as jnp
from jax import lax
from jax.experimental import pallas as pl
from jax.experimental.pallas import tpu as pltpu
from jax.experimental.pallas import tpu_sc as plsc

RAD_D = 60
OUT_D = 64
LANES = 16
NUM_WORKERS = 32
CHUNK = 128
NBUF = 4
LOOKAHEAD = 3


def _rsqrt16(x):
    y = plsc.bitcast(0x5F3759DF - (plsc.bitcast(x, jnp.int32) >> 1),
                     jnp.float32)
    for _ in range(3):
        y = y * (1.5 - 0.5 * x * y * y)
    return y


def _make_sc_call(n_b, n_l):
    n_rows = n_b * n_l
    rows_per_w = n_rows // NUM_WORKERS
    n_chunks = rows_per_w // CHUNK
    blk_per_w = rows_per_w // CHUNK
    assert rows_per_w % CHUNK == 0 and n_b % CHUNK == 0
    assert n_chunks % NBUF == 0 and NBUF > LOOKAHEAD

    mesh = plsc.VectorSubcoreMesh(core_axis_name="c", subcore_axis_name="s")

    @functools.partial(
        pl.kernel,
        out_type=jax.ShapeDtypeStruct((n_rows * OUT_D,), jnp.float32),
        mesh=mesh,
        compiler_params=pltpu.CompilerParams(
            needs_layout_passes=False, use_tc_tiling_on_sc=False),
        scratch_types=[
            pltpu.VMEM((rows_per_w,), jnp.int32),
            [pltpu.VMEM((CHUNK, OUT_D), jnp.float32)] * NBUF,
            [pltpu.VMEM((OUT_D * CHUNK,), jnp.float32)] * NBUF,
            [pltpu.SemaphoreType.DMA] * NBUF,
            [pltpu.SemaphoreType.DMA] * NBUF,
            [pltpu.SemaphoreType.DMA] * NBUF,
        ],
    )
    def sc_call(idx_hbm, pos_hbm, table_hbm, out_hbm, idx_v, gtiles, otiles,
                gsems, psems, osems):
        wid = lax.axis_index("s") * 2 + lax.axis_index("c")
        base = wid * rows_per_w
        iota = lax.iota(jnp.int32, LANES)

        pltpu.sync_copy(
            idx_hbm.at[pl.ds(pl.multiple_of(wid * rows_per_w, rows_per_w),
                             rows_per_w)],
            idx_v)

        def fire(ci, b):
            row0 = pl.multiple_of(base + ci * CHUNK, CHUNK)
            li = row0 // n_b
            b0 = row0 - li * n_b
            pltpu.async_copy(
                table_hbm.at[idx_v.at[pl.ds(ci * CHUNK, CHUNK)]],
                gtiles[b], gsems[b])
            for c in range(4):
                pltpu.async_copy(
                    pos_hbm.at[pl.ds((li * 4 + c) * n_b + b0, CHUNK)],
                    otiles[b].at[pl.ds((RAD_D + c) * CHUNK, CHUNK)],
                    psems[b])

        def wait_in(b):
            pltpu.make_async_copy(
                table_hbm.at[idx_v.at[pl.ds(0, CHUNK)]], gtiles[b],
                gsems[b]).wait()
            for c in range(4):
                pltpu.make_async_copy(
                    pos_hbm.at[pl.ds(0, CHUNK)],
                    otiles[b].at[pl.ds((RAD_D + c) * CHUNK, CHUNK)],
                    psems[b]).wait()

        TILE_W = 8 * CHUNK
        ROW_STRIDE = n_b * 8

        def wait_out(b):
            for k in range(OUT_D // 8):
                pltpu.make_async_copy(
                    otiles[b].at[pl.ds(k * TILE_W, TILE_W)],
                    out_hbm.at[pl.ds(k * TILE_W, TILE_W)],
                    osems[b]).wait()

        for p in range(LOOKAHEAD):
            fire(p, p)

        def quad_body(qi, carry):
            for b in range(NBUF):
                ci = qi * NBUF + b
                wait_in(b)
                gtile = gtiles[b]
                otile = otiles[b]

                n_grp = CHUNK // LANES
                rowids = [g * LANES + iota for g in range(n_grp)]
                UNROLL = 4

                def sumsq_body(ji, carry2):
                    cj = carry2[0]
                    accs = list(carry2[1:])
                    for _ in range(UNROLL):
                        for g in range(n_grp):
                            v = plsc.load_gather(gtile, [rowids[g], cj])
                            accs[g] = accs[g] + v * v
                        cj = (cj + 1) & 63
                    return (cj, *accs)

                zero = jnp.zeros((LANES,), jnp.float32)
                res = lax.fori_loop(0, OUT_D // UNROLL, sumsq_body,
                                    (iota,) + (zero,) * n_grp)
                scales = [
                    jnp.minimum(jnp.float32(1.0),
                                _rsqrt16(jnp.maximum(a, 1e-30)))
                    for a in res[1:]
                ]

                def scale_body(ji, cj):
                    for _ in range(UNROLL):
                        mj = cj < RAD_D
                        cshift = cj * CHUNK
                        for g in range(n_grp):
                            v = plsc.load_gather(gtile, [rowids[g], cj])
                            plsc.store_scatter(otile, [cshift + rowids[g]],
                                               v * scales[g], mask=mj)
                        cj = (cj + 1) & 63
                    return cj

                lax.fori_loop(0, OUT_D // UNROLL, scale_body, iota)

                row0 = pl.multiple_of(base + ci * CHUNK, CHUNK)
                li = row0 // n_b
                b0 = row0 - li * n_b
                for k in range(OUT_D // 8):
                    pltpu.async_copy(
                        otile.at[pl.ds(k * TILE_W, TILE_W)],
                        out_hbm.at[pl.ds((li * 8 + k) * ROW_STRIDE + b0 * 8,
                                         TILE_W)],
                        osems[b])

                nb = (b + LOOKAHEAD) % NBUF
                nci = ci + LOOKAHEAD

                @pl.when(nci < n_chunks)
                def _():
                    @pl.when(nci >= NBUF)
                    def _():
                        wait_out(nb)
                    fire(nci, nb)
            return carry

        lax.fori_loop(0, n_chunks // NBUF, quad_body, 0)
        for b in range(NBUF):
            wait_out(b)

    return sc_call


def kernel(indices, positions, table):
    n_b, n_l = indices.shape
    n_rows = n_b * n_l
    table64 = jnp.concatenate(
        [table, jnp.zeros((table.shape[0], OUT_D - RAD_D), table.dtype)],
        axis=1)
    out = _make_sc_call(n_b, n_l)(
        indices.T.reshape(n_rows),
        positions.transpose(1, 2, 0).reshape(n_rows * 4),
        table64,
    )
    return (out.reshape(n_l, 8, n_b // 128, 8, 128)
            .transpose(2, 4, 0, 1, 3)
            .reshape(n_b, n_l, OUT_D))


# speedup vs baseline: 1.0051x; 1.0051x over previous
import functools

import jax
import jax.numpy as jnp
from jax import lax
from jax.experimental import pallas as pl
from jax.experimental.pallas import tpu as pltpu
from jax.experimental.pallas import tpu_sc as plsc

RAD_D = 60
OUT_D = 64
LANES = 16
NUM_WORKERS = 32
CHUNK = 128
NBUF = 4
LOOKAHEAD = 3


def _rsqrt16(x):
    y = plsc.bitcast(0x5F3759DF - (plsc.bitcast(x, jnp.int32) >> 1),
                     jnp.float32)
    for _ in range(3):
        y = y * (1.5 - 0.5 * x * y * y)
    return y


def _make_sc_call(n_b, n_l):
    n_rows = n_b * n_l
    rows_per_w = n_rows // NUM_WORKERS
    n_chunks = rows_per_w // CHUNK
    blk_per_w = rows_per_w // CHUNK
    assert rows_per_w % CHUNK == 0 and n_b % CHUNK == 0
    assert n_chunks % NBUF == 0 and NBUF > LOOKAHEAD

    mesh = plsc.VectorSubcoreMesh(core_axis_name="c", subcore_axis_name="s")

    @functools.partial(
        pl.kernel,
        out_type=jax.ShapeDtypeStruct((n_rows * OUT_D,), jnp.float32),
        mesh=mesh,
        compiler_params=pltpu.CompilerParams(
            needs_layout_passes=False, use_tc_tiling_on_sc=False),
        scratch_types=[
            pltpu.VMEM((rows_per_w,), jnp.int32),
            [pltpu.VMEM((CHUNK, OUT_D), jnp.float32)] * NBUF,
            [pltpu.VMEM((OUT_D * CHUNK,), jnp.float32)] * NBUF,
            [pltpu.SemaphoreType.DMA] * NBUF,
            [pltpu.SemaphoreType.DMA] * NBUF,
            [pltpu.SemaphoreType.DMA] * NBUF,
        ],
    )
    def sc_call(idx_hbm, pos_hbm, table_hbm, out_hbm, idx_v, gtiles, otiles,
                gsems, psems, osems):
        wid = lax.axis_index("s") * 2 + lax.axis_index("c")
        base = wid * rows_per_w
        iota = lax.iota(jnp.int32, LANES)

        pltpu.sync_copy(
            idx_hbm.at[pl.ds(pl.multiple_of(wid * rows_per_w, rows_per_w),
                             rows_per_w)],
            idx_v)

        def fire(ci, b):
            row0 = pl.multiple_of(base + ci * CHUNK, CHUNK)
            li = row0 // n_b
            b0 = row0 - li * n_b
            pltpu.async_copy(
                table_hbm.at[idx_v.at[pl.ds(ci * CHUNK, CHUNK)]],
                gtiles[b], gsems[b])
            for c in range(4):
                pltpu.async_copy(
                    pos_hbm.at[pl.ds((li * 4 + c) * n_b + b0, CHUNK)],
                    otiles[b].at[pl.ds((RAD_D + c) * CHUNK, CHUNK)],
                    psems[b])

        def wait_in(b):
            pltpu.make_async_copy(
                table_hbm.at[idx_v.at[pl.ds(0, CHUNK)]], gtiles[b],
                gsems[b]).wait()
            for c in range(4):
                pltpu.make_async_copy(
                    pos_hbm.at[pl.ds(0, CHUNK)],
                    otiles[b].at[pl.ds((RAD_D + c) * CHUNK, CHUNK)],
                    psems[b]).wait()

        TILE_W = 8 * CHUNK
        ROW_STRIDE = n_b * 8

        def wait_out(b):
            for k in range(OUT_D // 8):
                pltpu.make_async_copy(
                    otiles[b].at[pl.ds(k * TILE_W, TILE_W)],
                    out_hbm.at[pl.ds(k * TILE_W, TILE_W)],
                    osems[b]).wait()

        for p in range(LOOKAHEAD):
            fire(p, p)

        def quad_body(qi, carry):
            for b in range(NBUF):
                ci = qi * NBUF + b
                wait_in(b)
                gtile = gtiles[b]
                otile = otiles[b]

                n_grp = CHUNK // LANES
                rowids = [g * LANES + iota for g in range(n_grp)]
                UNROLL = 8

                def sumsq_body(ji, carry2):
                    cj = carry2[0]
                    accs = list(carry2[1:])
                    for _ in range(UNROLL):
                        for g in range(n_grp):
                            v = plsc.load_gather(gtile, [rowids[g], cj])
                            accs[g] = accs[g] + v * v
                        cj = (cj + 1) & 63
                    return (cj, *accs)

                zero = jnp.zeros((LANES,), jnp.float32)
                res = lax.fori_loop(0, OUT_D // UNROLL, sumsq_body,
                                    (iota,) + (zero,) * n_grp)
                scales = [
                    jnp.minimum(jnp.float32(1.0),
                                _rsqrt16(jnp.maximum(a, 1e-30)))
                    for a in res[1:]
                ]

                def scale_body(ji, cj):
                    for _ in range(UNROLL):
                        mj = cj < RAD_D
                        cshift = cj * CHUNK
                        for g in range(n_grp):
                            v = plsc.load_gather(gtile, [rowids[g], cj])
                            plsc.store_scatter(otile, [cshift + rowids[g]],
                                               v * scales[g], mask=mj)
                        cj = (cj + 1) & 63
                    return cj

                lax.fori_loop(0, OUT_D // UNROLL, scale_body, iota)

                row0 = pl.multiple_of(base + ci * CHUNK, CHUNK)
                li = row0 // n_b
                b0 = row0 - li * n_b
                for k in range(OUT_D // 8):
                    pltpu.async_copy(
                        otile.at[pl.ds(k * TILE_W, TILE_W)],
                        out_hbm.at[pl.ds((li * 8 + k) * ROW_STRIDE + b0 * 8,
                                         TILE_W)],
                        osems[b])

                nb = (b + LOOKAHEAD) % NBUF
                nci = ci + LOOKAHEAD

                @pl.when(nci < n_chunks)
                def _():
                    @pl.when(nci >= NBUF)
                    def _():
                        wait_out(nb)
                    fire(nci, nb)
            return carry

        lax.fori_loop(0, n_chunks // NBUF, quad_body, 0)
        for b in range(NBUF):
            wait_out(b)

    return sc_call


def kernel(indices, positions, table):
    n_b, n_l = indices.shape
    n_rows = n_b * n_l
    table64 = jnp.pad(table, ((0, 0), (0, OUT_D - RAD_D)))
    out = _make_sc_call(n_b, n_l)(
        indices.T.reshape(n_rows),
        positions.transpose(1, 2, 0).reshape(n_rows * 4),
        table64,
    )
    return (out.reshape(n_l, 8, n_b // 128, 8, 128)
            .transpose(2, 4, 0, 1, 3)
            .reshape(n_b, n_l, OUT_D))
